# baseline (speedup 1.0000x reference)
import functools

import jax
import jax.numpy as jnp
from jax import lax
from jax.experimental import pallas as pl
from jax.experimental.pallas import tpu as pltpu
from jax.experimental.pallas import tpu_sc as plsc


_GATHER_CHUNK = 64


@functools.lru_cache(maxsize=None)
def _make_gather(vocab, dim, b_padded):
    info = plsc.get_sparse_core_info()
    nc, ns = info.num_cores, info.num_subcores
    nw = nc * ns
    b_per_w = b_padded // nw
    n_chunks = b_per_w // _GATHER_CHUNK
    mesh = plsc.VectorSubcoreMesh(core_axis_name="c", subcore_axis_name="s")

    @functools.partial(
        pl.kernel,
        mesh=mesh,
        out_type=jax.ShapeDtypeStruct((b_padded, dim), jnp.float32),
        scratch_types=[
            pltpu.VMEM((n_chunks, _GATHER_CHUNK), jnp.int32),
            pltpu.VMEM((b_per_w, dim), jnp.float32),
            pltpu.SemaphoreType.DMA,
        ],
    )
    def gather(table_hbm, idx_hbm, out_hbm, idx_v, rows_v, sem):
        wid = lax.axis_index("s") * nc + lax.axis_index("c")
        base = wid * b_per_w
        for j in range(n_chunks):
            pltpu.sync_copy(
                idx_hbm.at[pl.ds(base + j * _GATHER_CHUNK, _GATHER_CHUNK)],
                idx_v.at[j],
            )
        copies = []
        for j in range(n_chunks):
            copies.append(
                pltpu.async_copy(
                    table_hbm.at[idx_v.at[j]],
                    rows_v.at[pl.ds(j * _GATHER_CHUNK, _GATHER_CHUNK)],
                    sem,
                )
            )
        for c in copies:
            c.wait()
        pltpu.sync_copy(rows_v, out_hbm.at[pl.ds(base, b_per_w)])

    return gather


_BLK = 256


def _layer0_body(
    x0t_ref, x0t16_ref, adj_ref, wt_ref, bt_ref,
    x1t_ref, x1t16_ref, at16_ref, inv_ref, y16_ref,
):
    r = pl.program_id(0)
    n = adj_ref.shape[1]
    np_ = x0t_ref.shape[1]

    @pl.when(r == 0)
    def _():
        y = jnp.dot(wt_ref[0], x0t_ref[...], preferred_element_type=jnp.float32)
        y16_ref[...] = y.astype(jnp.bfloat16)

    a = adj_ref[...]
    a16p = jnp.concatenate(
        [a.astype(jnp.bfloat16), jnp.zeros((_BLK, np_ - n), jnp.bfloat16)], axis=1
    )
    at16 = a16p.T
    at16_ref[...] = at16
    deg = jnp.sum(at16.astype(jnp.float32), axis=0, keepdims=True)
    iv = 1.0 / (deg + 1e-6)
    inv_ref[...] = iv
    aggt = jnp.dot(y16_ref[...], at16, preferred_element_type=jnp.float32)
    h = jnp.maximum(aggt * iv + bt_ref[0], 0.0) + x0t_ref[:, pl.ds(r * _BLK, _BLK)]
    dim = h.shape[0]
    node = lax.broadcasted_iota(jnp.int32, (dim, _BLK), 1) + r * _BLK
    h = jnp.where(node < n, h, 0.0)
    x1t_ref[...] = h
    x1t16_ref[...] = h.astype(jnp.bfloat16)


def _layer0(x0t, x0t16, adj, wt, bt):
    dim, np_ = x0t.shape
    n = adj.shape[0]
    layers = wt.shape[0]
    nb = np_ // _BLK
    return pl.pallas_call(
        _layer0_body,
        grid=(nb,),
        in_specs=[
            pl.BlockSpec((dim, np_), lambda r: (0, 0)),
            pl.BlockSpec((dim, np_), lambda r: (0, 0)),
            pl.BlockSpec((_BLK, n), lambda r: (r, 0)),
            pl.BlockSpec((layers, dim, dim), lambda r: (0, 0, 0)),
            pl.BlockSpec((layers, dim, 1), lambda r: (0, 0, 0)),
        ],
        out_specs=[
            pl.BlockSpec((dim, _BLK), lambda r: (0, r)),
            pl.BlockSpec((dim, _BLK), lambda r: (0, r)),
            pl.BlockSpec((np_, _BLK), lambda r: (0, r)),
            pl.BlockSpec((1, _BLK), lambda r: (0, r)),
        ],
        out_shape=[
            jax.ShapeDtypeStruct((dim, np_), jnp.float32),
            jax.ShapeDtypeStruct((dim, np_), jnp.bfloat16),
            jax.ShapeDtypeStruct((np_, np_), jnp.bfloat16),
            jax.ShapeDtypeStruct((1, np_), jnp.float32),
        ],
        scratch_shapes=[
            pltpu.VMEM((dim, np_), jnp.bfloat16),
        ],
        compiler_params=pltpu.CompilerParams(
            dimension_semantics=("arbitrary",),
        ),
    )(x0t, x0t16, adj, wt, bt)


def _rest_body(
    n_real, x1t_ref, x1t16_ref, at16_ref, inv_ref, wt_ref, bt_ref, out_ref,
    xta, xta16, xtb, xtb16, y16_ref,
):
    li = pl.program_id(0)
    r = pl.program_id(1)
    at16 = at16_ref[...]
    iv = inv_ref[:, pl.ds(r * _BLK, _BLK)]
    bl = bt_ref[li + 1]

    def compute_y(src_t_ref):
        y = jnp.dot(
            wt_ref[li + 1], src_t_ref[...], preferred_element_type=jnp.float32
        )
        y16_ref[...] = y.astype(jnp.bfloat16)

    def step(res_t_ref):
        aggt = jnp.dot(y16_ref[...], at16, preferred_element_type=jnp.float32)
        lin = aggt * iv + bl
        h = jnp.maximum(lin, 0.0) + res_t_ref[:, pl.ds(r * _BLK, _BLK)]
        dim = h.shape[0]
        node = lax.broadcasted_iota(jnp.int32, (dim, _BLK), 1) + r * _BLK
        return jnp.where(node < n_real, h, 0.0)

    @pl.when(li == 0)
    def _():
        @pl.when(r == 0)
        def _():
            compute_y(x1t_ref)

        h = step(x1t_ref)
        xta[:, pl.ds(r * _BLK, _BLK)] = h
        xta16[:, pl.ds(r * _BLK, _BLK)] = h.astype(jnp.bfloat16)

    @pl.when(li == 1)
    def _():
        @pl.when(r == 0)
        def _():
            compute_y(xta)

        h = step(xta)
        xtb[:, pl.ds(r * _BLK, _BLK)] = h
        xtb16[:, pl.ds(r * _BLK, _BLK)] = h.astype(jnp.bfloat16)

    @pl.when(li == 2)
    def _():
        @pl.when(r == 0)
        def _():
            compute_y(xtb)

        h = step(xtb)
        dim = h.shape[0]
        node = lax.broadcasted_iota(jnp.int32, (dim, _BLK), 1) + r * _BLK
        hm = jnp.where(node < n_real, h, -jnp.inf)
        m = jnp.max(hm, axis=1, keepdims=True)

        @pl.when(r == 0)
        def _():
            out_ref[...] = m

        @pl.when(r > 0)
        def _():
            out_ref[...] = jnp.maximum(out_ref[...], m)


def _rest(n_real, x1t, x1t16, at16, inv, wt, bt):
    dim, np_ = x1t.shape
    layers = wt.shape[0]
    nb = np_ // _BLK
    return pl.pallas_call(
        functools.partial(_rest_body, n_real),
        grid=(layers - 1, nb),
        in_specs=[
            pl.BlockSpec((dim, np_), lambda l, r: (0, 0)),
            pl.BlockSpec((dim, np_), lambda l, r: (0, 0)),
            pl.BlockSpec((np_, _BLK), lambda l, r: (0, r)),
            pl.BlockSpec((1, np_), lambda l, r: (0, 0)),
            pl.BlockSpec((layers, dim, dim), lambda l, r: (0, 0, 0)),
            pl.BlockSpec((layers, dim, 1), lambda l, r: (0, 0, 0)),
        ],
        out_specs=pl.BlockSpec((dim, 1), lambda l, r: (0, 0)),
        out_shape=jax.ShapeDtypeStruct((dim, 1), jnp.float32),
        scratch_shapes=[
            pltpu.VMEM((dim, np_), jnp.float32),
            pltpu.VMEM((dim, np_), jnp.bfloat16),
            pltpu.VMEM((dim, np_), jnp.float32),
            pltpu.VMEM((dim, np_), jnp.bfloat16),
            pltpu.VMEM((dim, np_), jnp.bfloat16),
        ],
        compiler_params=pltpu.CompilerParams(
            dimension_semantics=("arbitrary", "arbitrary"),
        ),
    )(x1t, x1t16, at16, inv, wt, bt)


def kernel(token_ids, adj, emb, W, b):
    n = adj.shape[0]
    vocab, dim = emb.shape

    info = plsc.get_sparse_core_info()
    nw = info.num_cores * info.num_subcores
    quantum = max(nw * _GATHER_CHUNK, _BLK)
    n_pad = ((n + quantum - 1) // quantum) * quantum
    ids = jnp.pad(token_ids.astype(jnp.int32), (0, n_pad - n))
    x0 = _make_gather(vocab, dim, n_pad)(emb, ids)

    x0t = x0.T
    x0t16 = x0t.astype(jnp.bfloat16)
    wt = W.transpose(0, 2, 1)
    bt = b[..., None]

    x1t, x1t16, at16, inv = _layer0(x0t, x0t16, adj, wt, bt)
    pooled = _rest(n, x1t, x1t16, at16, inv, wt, bt)
    return pooled.reshape(dim)


# speedup vs baseline: 1.0295x; 1.0295x over previous
import functools

import jax
import jax.numpy as jnp
from jax import lax
from jax.experimental import pallas as pl
from jax.experimental.pallas import tpu as pltpu
from jax.experimental.pallas import tpu_sc as plsc


_GATHER_CHUNK = 64


@functools.lru_cache(maxsize=None)
def _make_gather(vocab, dim, b_padded):
    info = plsc.get_sparse_core_info()
    nc, ns = info.num_cores, info.num_subcores
    nw = nc * ns
    b_per_w = b_padded // nw
    n_chunks = b_per_w // _GATHER_CHUNK
    mesh = plsc.VectorSubcoreMesh(core_axis_name="c", subcore_axis_name="s")

    @functools.partial(
        pl.kernel,
        mesh=mesh,
        out_type=jax.ShapeDtypeStruct((b_padded, dim), jnp.float32),
        scratch_types=[
            pltpu.VMEM((n_chunks, _GATHER_CHUNK), jnp.int32),
            pltpu.VMEM((b_per_w, dim), jnp.float32),
            pltpu.SemaphoreType.DMA,
        ],
    )
    def gather(table_hbm, idx_hbm, out_hbm, idx_v, rows_v, sem):
        wid = lax.axis_index("s") * nc + lax.axis_index("c")
        base = wid * b_per_w
        for j in range(n_chunks):
            pltpu.sync_copy(
                idx_hbm.at[pl.ds(base + j * _GATHER_CHUNK, _GATHER_CHUNK)],
                idx_v.at[j],
            )
        copies = []
        for j in range(n_chunks):
            copies.append(
                pltpu.async_copy(
                    table_hbm.at[idx_v.at[j]],
                    rows_v.at[pl.ds(j * _GATHER_CHUNK, _GATHER_CHUNK)],
                    sem,
                )
            )
        for c in copies:
            c.wait()
        pltpu.sync_copy(rows_v, out_hbm.at[pl.ds(base, b_per_w)])

    return gather


def _layer0_body(x0_ref, adj_ref, w_ref, b_ref, x1_ref, adj16_ref, inv_ref):
    a = adj_ref[...]
    deg = jnp.sum(a, axis=1, keepdims=True)
    iv = 1.0 / (deg + 1e-6)
    inv_ref[...] = iv
    a16 = a.astype(jnp.bfloat16)
    adj16_ref[...] = a16
    x = x0_ref[...]
    agg = jnp.dot(a16, x.astype(jnp.bfloat16), preferred_element_type=jnp.float32)
    agg = agg * iv
    lin = jnp.dot(agg, w_ref[0], preferred_element_type=jnp.float32) + b_ref[0]
    blk = adj_ref.shape[0]
    r = pl.program_id(0)
    x1_ref[...] = jnp.maximum(lin, 0.0) + x0_ref[pl.ds(r * blk, blk), :]


def _layer0(x0, adj, w, b, blk):
    n, dim = x0.shape
    layers = w.shape[0]
    nb = n // blk
    return pl.pallas_call(
        _layer0_body,
        grid=(nb,),
        in_specs=[
            pl.BlockSpec((n, dim), lambda r: (0, 0)),
            pl.BlockSpec((blk, n), lambda r: (r, 0)),
            pl.BlockSpec((layers, dim, dim), lambda r: (0, 0, 0)),
            pl.BlockSpec((layers, dim), lambda r: (0, 0)),
        ],
        out_specs=[
            pl.BlockSpec((blk, dim), lambda r: (r, 0)),
            pl.BlockSpec((blk, n), lambda r: (r, 0)),
            pl.BlockSpec((blk, 1), lambda r: (r, 0)),
        ],
        out_shape=[
            jax.ShapeDtypeStruct((n, dim), jnp.float32),
            jax.ShapeDtypeStruct((n, n), jnp.bfloat16),
            jax.ShapeDtypeStruct((n, 1), jnp.float32),
        ],
        compiler_params=pltpu.CompilerParams(
            dimension_semantics=("arbitrary",),
        ),
    )(x0, adj, w, b)


def _rest_body(x1_ref, adj16_ref, inv_ref, w_ref, b_ref, out_ref, xa, xb):
    li = pl.program_id(0)
    r = pl.program_id(1)
    blk = adj16_ref.shape[0]
    a16 = adj16_ref[...]
    iv = inv_ref[pl.ds(r * blk, blk), :]
    wl = w_ref[li + 1]
    bl = b_ref[li + 1]

    def step(src_ref):
        x = src_ref[...]
        agg = jnp.dot(
            a16, x.astype(jnp.bfloat16), preferred_element_type=jnp.float32
        ) * iv
        lin = jnp.dot(agg, wl, preferred_element_type=jnp.float32) + bl
        return jnp.maximum(lin, 0.0) + src_ref[pl.ds(r * blk, blk), :]

    @pl.when(li == 0)
    def _():
        xa[pl.ds(r * blk, blk), :] = step(x1_ref)

    @pl.when(li == 1)
    def _():
        xb[pl.ds(r * blk, blk), :] = step(xa)

    @pl.when(li == 2)
    def _():
        h = step(xb)
        m = jnp.max(h, axis=0, keepdims=True)

        @pl.when(r == 0)
        def _():
            out_ref[...] = m

        @pl.when(r > 0)
        def _():
            out_ref[...] = jnp.maximum(out_ref[...], m)


def _rest(x1, adj16, inv, w, b, blk):
    n, dim = x1.shape
    layers = w.shape[0]
    nb = n // blk
    return pl.pallas_call(
        _rest_body,
        grid=(layers - 1, nb),
        in_specs=[
            pl.BlockSpec((n, dim), lambda l, r: (0, 0)),
            pl.BlockSpec((blk, n), lambda l, r: (r, 0)),
            pl.BlockSpec((n, 1), lambda l, r: (0, 0)),
            pl.BlockSpec((layers, dim, dim), lambda l, r: (0, 0, 0)),
            pl.BlockSpec((layers, dim), lambda l, r: (0, 0)),
        ],
        out_specs=pl.BlockSpec((1, dim), lambda l, r: (0, 0)),
        out_shape=jax.ShapeDtypeStruct((1, dim), jnp.float32),
        scratch_shapes=[
            pltpu.VMEM((n, dim), jnp.float32),
            pltpu.VMEM((n, dim), jnp.float32),
        ],
        compiler_params=pltpu.CompilerParams(
            dimension_semantics=("arbitrary", "arbitrary"),
        ),
    )(x1, adj16, inv, w, b)


def kernel(token_ids, adj, emb, W, b):
    n = adj.shape[0]
    vocab, dim = emb.shape

    info = plsc.get_sparse_core_info()
    nw = info.num_cores * info.num_subcores
    quantum = nw * _GATHER_CHUNK
    b_padded = ((n + quantum - 1) // quantum) * quantum
    ids = jnp.pad(token_ids.astype(jnp.int32), (0, b_padded - n))
    x0 = _make_gather(vocab, dim, b_padded)(emb, ids)[:n]

    x1, adj16, inv = _layer0(x0, adj, W, b, blk=200)
    pooled = _rest(x1, adj16, inv, W, b, blk=400)
    return pooled.reshape(dim)


# speedup vs baseline: 1.0395x; 1.0097x over previous
import functools

import jax
import jax.numpy as jnp
from jax import lax
from jax.experimental import pallas as pl
from jax.experimental.pallas import tpu as pltpu
from jax.experimental.pallas import tpu_sc as plsc


_GATHER_CHUNK = 64


@functools.lru_cache(maxsize=None)
def _make_gather(vocab, dim, b_padded):
    info = plsc.get_sparse_core_info()
    nc, ns = info.num_cores, info.num_subcores
    nw = nc * ns
    b_per_w = b_padded // nw
    n_chunks = b_per_w // _GATHER_CHUNK
    mesh = plsc.VectorSubcoreMesh(core_axis_name="c", subcore_axis_name="s")

    @functools.partial(
        pl.kernel,
        mesh=mesh,
        out_type=jax.ShapeDtypeStruct((b_padded, dim), jnp.float32),
        scratch_types=[
            pltpu.VMEM((n_chunks, _GATHER_CHUNK), jnp.int32),
            pltpu.VMEM((b_per_w, dim), jnp.float32),
            pltpu.SemaphoreType.DMA,
        ],
    )
    def gather(table_hbm, idx_hbm, out_hbm, idx_v, rows_v, sem):
        wid = lax.axis_index("s") * nc + lax.axis_index("c")
        base = wid * b_per_w
        for j in range(n_chunks):
            pltpu.sync_copy(
                idx_hbm.at[pl.ds(base + j * _GATHER_CHUNK, _GATHER_CHUNK)],
                idx_v.at[j],
            )
        copies = []
        for j in range(n_chunks):
            copies.append(
                pltpu.async_copy(
                    table_hbm.at[idx_v.at[j]],
                    rows_v.at[pl.ds(j * _GATHER_CHUNK, _GATHER_CHUNK)],
                    sem,
                )
            )
        for c in copies:
            c.wait()
        pltpu.sync_copy(rows_v, out_hbm.at[pl.ds(base, b_per_w)])

    return gather


def _layer0_body(x0_ref, adj_ref, w_ref, b_ref, x1_ref, adj16_ref, inv_ref):
    a = adj_ref[...]
    deg = jnp.sum(a, axis=1, keepdims=True)
    iv = 1.0 / (deg + 1e-6)
    inv_ref[...] = iv
    a16 = a.astype(jnp.bfloat16)
    adj16_ref[...] = a16
    x = x0_ref[...]
    agg = jnp.dot(a16, x.astype(jnp.bfloat16), preferred_element_type=jnp.float32)
    agg = agg * iv
    lin = jnp.dot(agg, w_ref[0], preferred_element_type=jnp.float32) + b_ref[0]
    blk = adj_ref.shape[0]
    r = pl.program_id(0)
    x1_ref[...] = jnp.maximum(lin, 0.0) + x0_ref[pl.ds(r * blk, blk), :]


def _layer0(x0, adj, w, b, blk):
    n, dim = x0.shape
    layers = w.shape[0]
    nb = n // blk
    return pl.pallas_call(
        _layer0_body,
        grid=(nb,),
        in_specs=[
            pl.BlockSpec((n, dim), lambda r: (0, 0)),
            pl.BlockSpec((blk, n), lambda r: (r, 0)),
            pl.BlockSpec((layers, dim, dim), lambda r: (0, 0, 0)),
            pl.BlockSpec((layers, dim), lambda r: (0, 0)),
        ],
        out_specs=[
            pl.BlockSpec((blk, dim), lambda r: (r, 0)),
            pl.BlockSpec((blk, n), lambda r: (r, 0)),
            pl.BlockSpec((blk, 1), lambda r: (r, 0)),
        ],
        out_shape=[
            jax.ShapeDtypeStruct((n, dim), jnp.float32),
            jax.ShapeDtypeStruct((n, n), jnp.bfloat16),
            jax.ShapeDtypeStruct((n, 1), jnp.float32),
        ],
        compiler_params=pltpu.CompilerParams(
            dimension_semantics=("arbitrary",),
        ),
    )(x0, adj, w, b)


def _rest_body(x1_ref, adj16_ref, inv_ref, w_ref, b_ref, out_ref, xa, xb):
    li = pl.program_id(0)
    r = pl.program_id(1)
    blk = adj16_ref.shape[0]
    a16 = adj16_ref[...]
    iv = inv_ref[pl.ds(r * blk, blk), :]
    wl = w_ref[li + 1]
    bl = b_ref[li + 1]

    def step(src_ref):
        x = src_ref[...]
        agg = jnp.dot(
            a16, x.astype(jnp.bfloat16), preferred_element_type=jnp.float32
        ) * iv
        lin = jnp.dot(agg, wl, preferred_element_type=jnp.float32) + bl
        return jnp.maximum(lin, 0.0) + src_ref[pl.ds(r * blk, blk), :]

    @pl.when(li == 0)
    def _():
        xa[pl.ds(r * blk, blk), :] = step(x1_ref)

    @pl.when(li == 1)
    def _():
        xb[pl.ds(r * blk, blk), :] = step(xa)

    @pl.when(li == 2)
    def _():
        h = step(xb)
        m = jnp.max(h, axis=0, keepdims=True)

        @pl.when(r == 0)
        def _():
            out_ref[...] = m

        @pl.when(r > 0)
        def _():
            out_ref[...] = jnp.maximum(out_ref[...], m)


def _rest(x1, adj16, inv, w, b, blk):
    n, dim = x1.shape
    layers = w.shape[0]
    nb = n // blk
    return pl.pallas_call(
        _rest_body,
        grid=(layers - 1, nb),
        in_specs=[
            pl.BlockSpec((n, dim), lambda l, r: (0, 0)),
            pl.BlockSpec((blk, n), lambda l, r: (r, 0)),
            pl.BlockSpec((n, 1), lambda l, r: (0, 0)),
            pl.BlockSpec((layers, dim, dim), lambda l, r: (0, 0, 0)),
            pl.BlockSpec((layers, dim), lambda l, r: (0, 0)),
        ],
        out_specs=pl.BlockSpec((1, dim), lambda l, r: (0, 0)),
        out_shape=jax.ShapeDtypeStruct((1, dim), jnp.float32),
        scratch_shapes=[
            pltpu.VMEM((n, dim), jnp.float32),
            pltpu.VMEM((n, dim), jnp.float32),
        ],
        compiler_params=pltpu.CompilerParams(
            dimension_semantics=("arbitrary", "arbitrary"),
        ),
    )(x1, adj16, inv, w, b)


def kernel(token_ids, adj, emb, W, b):
    n = adj.shape[0]
    vocab, dim = emb.shape

    info = plsc.get_sparse_core_info()
    nw = info.num_cores * info.num_subcores
    quantum = nw * _GATHER_CHUNK
    b_padded = ((n + quantum - 1) // quantum) * quantum
    ids = jnp.pad(token_ids.astype(jnp.int32), (0, b_padded - n))
    x0 = _make_gather(vocab, dim, b_padded)(emb, ids)[:n]

    x1, adj16, inv = _layer0(x0, adj, W, b, blk=400)
    pooled = _rest(x1, adj16, inv, W, b, blk=400)
    return pooled.reshape(dim)
